# Initial kernel scaffold; baseline (speedup 1.0000x reference)
import functools
import math

import jax
import jax.numpy as jnp
from jax import lax
from jax.experimental import pallas as pl
from jax.experimental.pallas import tpu as pltpu

N = 10000
E = 320000
D_IN = 128
HID = 64
HEADS = 4
N_CLASSES = 16

_ROW_BLK = 1000


def _proj1_body(x_ref, w_ref, s_ref, h_ref, aa_ref):
    h = jnp.dot(x_ref[...], w_ref[...], preferred_element_type=jnp.float32)
    h_ref[...] = h
    aa_ref[...] = jnp.dot(h, s_ref[...], preferred_element_type=jnp.float32)


def _proj1(x, W1, S1):
    grid = (N // _ROW_BLK,)
    return pl.pallas_call(
        _proj1_body,
        grid=grid,
        in_specs=[
            pl.BlockSpec((_ROW_BLK, D_IN), lambda i: (i, 0)),
            pl.BlockSpec((D_IN, HEADS * HID), lambda i: (0, 0)),
            pl.BlockSpec((HEADS * HID, 16), lambda i: (0, 0)),
        ],
        out_specs=[
            pl.BlockSpec((_ROW_BLK, HEADS * HID), lambda i: (i, 0)),
            pl.BlockSpec((_ROW_BLK, 16), lambda i: (i, 0)),
        ],
        out_shape=[
            jax.ShapeDtypeStruct((N, HEADS * HID), jnp.float32),
            jax.ShapeDtypeStruct((N, 16), jnp.float32),
        ],
    )(x, W1, S1)


def _leaky(x):
    return jnp.where(x > 0, x, 0.2 * x)


def _elu(x):
    return jnp.where(x > 0, x, jnp.expm1(x))


def _finalize1_body(p0_ref, p1_ref, h1_ref, aa_ref, b1_ref, g1_ref, be1_ref,
                    w2_ref, s2_ref, h2_ref, aa2_ref):
    aa = aa_ref[...]
    ex_ii = jnp.exp(_leaky(aa[:, 0:4] + aa[:, 8:12]))
    h1 = h1_ref[...]
    cols = []
    for h in range(HEADS):
        part = p0_ref[...] if h < 2 else p1_ref[...]
        j = h % 2
        num = part[:, j * HID:(j + 1) * HID] + ex_ii[:, h:h + 1] * h1[:, h * HID:(h + 1) * HID]
        den = part[:, 128 + j:129 + j] + ex_ii[:, h:h + 1]
        cols.append(num / (den + 1e-16))
    agg = jnp.concatenate(cols, axis=1) + b1_ref[...]
    scale = g1_ref[...] * (1.0 / math.sqrt(1.0 + 1e-5))
    act = _elu(agg * scale + be1_ref[...])
    h2 = jnp.dot(act, w2_ref[...], preferred_element_type=jnp.float32)
    h2_ref[...] = h2
    aa2_ref[...] = jnp.dot(h2, s2_ref[...], preferred_element_type=jnp.float32)


def _finalize1(p0, p1, h1, aa16, b1, g1, be1, W2, S2):
    F = HEADS * HID
    grid = (N // _ROW_BLK,)
    row = lambda i: (i, 0)
    full = lambda i: (0, 0)
    return pl.pallas_call(
        _finalize1_body,
        grid=grid,
        in_specs=[
            pl.BlockSpec((_ROW_BLK, 144), row),
            pl.BlockSpec((_ROW_BLK, 144), row),
            pl.BlockSpec((_ROW_BLK, F), row),
            pl.BlockSpec((_ROW_BLK, 16), row),
            pl.BlockSpec((1, F), full),
            pl.BlockSpec((1, F), full),
            pl.BlockSpec((1, F), full),
            pl.BlockSpec((F, HID), full),
            pl.BlockSpec((HID, 16), full),
        ],
        out_specs=[
            pl.BlockSpec((_ROW_BLK, HID), row),
            pl.BlockSpec((_ROW_BLK, 16), row),
        ],
        out_shape=[
            jax.ShapeDtypeStruct((N, HID), jnp.float32),
            jax.ShapeDtypeStruct((N, 16), jnp.float32),
        ],
    )(p0, p1, h1, aa16, b1, g1, be1, W2, S2)


def _finalize2_body(q0_ref, q1_ref, h2_ref, aa2_ref, b2_ref, g2_ref, be2_ref,
                    fcw_ref, fcb_ref, out_ref):
    aa2 = aa2_ref[...]
    ex_ii = jnp.exp(_leaky(aa2[:, 0:1] + aa2[:, 8:9]))
    h2 = h2_ref[...]
    q0 = q0_ref[...]
    q1 = q1_ref[...]
    num = q0[:, :HID] + q1[:, :HID] + ex_ii * h2
    den = q0[:, HID:HID + 1] + q1[:, HID:HID + 1] + ex_ii
    agg = num / (den + 1e-16) + b2_ref[...]
    scale = g2_ref[...] * (1.0 / math.sqrt(1.0 + 1e-5))
    act = _elu(agg * scale + be2_ref[...])
    logits = jnp.dot(act, fcw_ref[...], preferred_element_type=jnp.float32) + fcb_ref[...]
    m = jnp.max(logits, axis=1, keepdims=True)
    lse = m + jnp.log(jnp.sum(jnp.exp(logits - m), axis=1, keepdims=True))
    out_ref[...] = logits - lse


def _finalize2(q0, q1, h2, aa2, b2, g2, be2, fcW, fcb):
    grid = (N // _ROW_BLK,)
    row = lambda i: (i, 0)
    full = lambda i: (0, 0)
    return pl.pallas_call(
        _finalize2_body,
        grid=grid,
        in_specs=[
            pl.BlockSpec((_ROW_BLK, 80), row),
            pl.BlockSpec((_ROW_BLK, 80), row),
            pl.BlockSpec((_ROW_BLK, HID), row),
            pl.BlockSpec((_ROW_BLK, 16), row),
            pl.BlockSpec((1, HID), full),
            pl.BlockSpec((1, HID), full),
            pl.BlockSpec((1, HID), full),
            pl.BlockSpec((HID, N_CLASSES), full),
            pl.BlockSpec((1, N_CLASSES), full),
        ],
        out_specs=pl.BlockSpec((_ROW_BLK, N_CLASSES), row),
        out_shape=jax.ShapeDtypeStruct((N, N_CLASSES), jnp.float32),
    )(q0, q1, h2, aa2, b2, g2, be2, fcW, fcb)


def _edges_layer1_jnp(h1, aa16, src, dst):
    ex = jnp.exp(_leaky(aa16[src, 0:4] + aa16[dst, 8:12]))
    parts = []
    for c in range(2):
        hcols = h1[:, c * 128:(c + 1) * 128]
        w = jnp.repeat(ex[:, 2 * c:2 * c + 2], HID, axis=1)
        num = jax.ops.segment_sum(hcols[src] * w, dst, num_segments=N)
        den = jax.ops.segment_sum(ex[:, 2 * c:2 * c + 2], dst, num_segments=N)
        parts.append(jnp.concatenate(
            [num, den, jnp.zeros((N, 14), jnp.float32)], axis=1))
    return parts[0], parts[1]


def _edges_layer2_jnp(h2, aa2, src, dst):
    ex = jnp.exp(_leaky(aa2[src, 0:1] + aa2[dst, 8:9]))
    num = jax.ops.segment_sum(h2[src] * ex, dst, num_segments=N)
    den = jax.ops.segment_sum(ex, dst, num_segments=N)
    q0 = jnp.concatenate([num, den, jnp.zeros((N, 15), jnp.float32)], axis=1)
    q1 = jnp.zeros_like(q0)
    return q0, q1


def _att_proj_mat1(att_src1, att_dst1):
    eye = jnp.eye(HEADS, dtype=jnp.float32)
    ms = (att_src1[:, :, None] * eye[:, None, :]).reshape(HEADS * HID, HEADS)
    md = (att_dst1[:, :, None] * eye[:, None, :]).reshape(HEADS * HID, HEADS)
    z = jnp.zeros((HEADS * HID, 4), jnp.float32)
    return jnp.concatenate([ms, z, md, z], axis=1)


def _att_proj_mat2(att_src2, att_dst2):
    z = jnp.zeros((HID, 7), jnp.float32)
    return jnp.concatenate(
        [att_src2.T, z, att_dst2.T, z], axis=1)


def kernel(x, edge_index, W1, att_src1, att_dst1, b1, g1, be1,
           W2, att_src2, att_dst2, b2, g2, be2, fcW, fcb):
    src = edge_index[0]
    dst = edge_index[1]
    S1 = _att_proj_mat1(att_src1, att_dst1)
    S2 = _att_proj_mat2(att_src2, att_dst2)

    h1, aa16 = _proj1(x, W1, S1)
    p0, p1 = _edges_layer1_jnp(h1, aa16, src, dst)
    h2, aa2 = _finalize1(p0, p1, h1, aa16,
                         b1.reshape(1, -1), g1.reshape(1, -1),
                         be1.reshape(1, -1), W2, S2)
    q0, q1 = _edges_layer2_jnp(h2, aa2, src, dst)
    out = _finalize2(q0, q1, h2, aa2,
                     b2.reshape(1, -1), g2.reshape(1, -1),
                     be2.reshape(1, -1), fcW, fcb)
    return out


# speedup vs baseline: 1.0357x; 1.0357x over previous
import functools
import math

import jax
import jax.numpy as jnp
from jax import lax
from jax.experimental import pallas as pl
from jax.experimental.pallas import tpu as pltpu

N = 10000
E = 320000
D_IN = 128
HID = 64
HEADS = 4
N_CLASSES = 16

_ROW_BLK = 1000


def _proj1_body(x_ref, w_ref, s_ref, h_ref, aa_ref):
    h = jnp.dot(x_ref[...], w_ref[...], preferred_element_type=jnp.float32)
    h_ref[...] = h
    aa_ref[...] = jnp.dot(h, s_ref[...], preferred_element_type=jnp.float32)


def _proj1(x, W1, S1):
    grid = (N // _ROW_BLK,)
    return pl.pallas_call(
        _proj1_body,
        grid=grid,
        in_specs=[
            pl.BlockSpec((_ROW_BLK, D_IN), lambda i: (i, 0)),
            pl.BlockSpec((D_IN, HEADS * HID), lambda i: (0, 0)),
            pl.BlockSpec((HEADS * HID, 16), lambda i: (0, 0)),
        ],
        out_specs=[
            pl.BlockSpec((_ROW_BLK, HEADS * HID), lambda i: (i, 0)),
            pl.BlockSpec((_ROW_BLK, 16), lambda i: (i, 0)),
        ],
        out_shape=[
            jax.ShapeDtypeStruct((N, HEADS * HID), jnp.float32),
            jax.ShapeDtypeStruct((N, 16), jnp.float32),
        ],
    )(x, W1, S1)


def _leaky(x):
    return jnp.where(x > 0, x, 0.2 * x)


def _elu(x):
    return jnp.where(x > 0, x, jnp.exp(x) - 1.0)


def _finalize1_body(p0_ref, p1_ref, h1_ref, aa_ref, b1_ref, g1_ref, be1_ref,
                    w2_ref, s2_ref, h2_ref, aa2_ref):
    aa = aa_ref[...]
    ex_ii = jnp.exp(_leaky(aa[:, 0:4] + aa[:, 8:12]))
    h1 = h1_ref[...]
    cols = []
    for h in range(HEADS):
        part = p0_ref[...] if h < 2 else p1_ref[...]
        j = h % 2
        num = part[:, j * HID:(j + 1) * HID] + ex_ii[:, h:h + 1] * h1[:, h * HID:(h + 1) * HID]
        den = part[:, 128 + j:129 + j] + ex_ii[:, h:h + 1]
        cols.append(num / (den + 1e-16))
    agg = jnp.concatenate(cols, axis=1) + b1_ref[...]
    scale = g1_ref[...] * (1.0 / math.sqrt(1.0 + 1e-5))
    act = _elu(agg * scale + be1_ref[...])
    h2 = jnp.dot(act, w2_ref[...], preferred_element_type=jnp.float32)
    h2_ref[...] = h2
    aa2_ref[...] = jnp.dot(h2, s2_ref[...], preferred_element_type=jnp.float32)


def _finalize1(p0, p1, h1, aa16, b1, g1, be1, W2, S2):
    F = HEADS * HID
    grid = (N // _ROW_BLK,)
    row = lambda i: (i, 0)
    full = lambda i: (0, 0)
    return pl.pallas_call(
        _finalize1_body,
        grid=grid,
        in_specs=[
            pl.BlockSpec((_ROW_BLK, 144), row),
            pl.BlockSpec((_ROW_BLK, 144), row),
            pl.BlockSpec((_ROW_BLK, F), row),
            pl.BlockSpec((_ROW_BLK, 16), row),
            pl.BlockSpec((1, F), full),
            pl.BlockSpec((1, F), full),
            pl.BlockSpec((1, F), full),
            pl.BlockSpec((F, HID), full),
            pl.BlockSpec((HID, 16), full),
        ],
        out_specs=[
            pl.BlockSpec((_ROW_BLK, HID), row),
            pl.BlockSpec((_ROW_BLK, 16), row),
        ],
        out_shape=[
            jax.ShapeDtypeStruct((N, HID), jnp.float32),
            jax.ShapeDtypeStruct((N, 16), jnp.float32),
        ],
    )(p0, p1, h1, aa16, b1, g1, be1, W2, S2)


def _finalize2_body(q0_ref, q1_ref, h2_ref, aa2_ref, b2_ref, g2_ref, be2_ref,
                    fcw_ref, fcb_ref, out_ref):
    aa2 = aa2_ref[...]
    ex_ii = jnp.exp(_leaky(aa2[:, 0:1] + aa2[:, 8:9]))
    h2 = h2_ref[...]
    q0 = q0_ref[...]
    q1 = q1_ref[...]
    num = q0[:, :HID] + q1[:, :HID] + ex_ii * h2
    den = q0[:, HID:HID + 1] + q1[:, HID:HID + 1] + ex_ii
    agg = num / (den + 1e-16) + b2_ref[...]
    scale = g2_ref[...] * (1.0 / math.sqrt(1.0 + 1e-5))
    act = _elu(agg * scale + be2_ref[...])
    logits = jnp.dot(act, fcw_ref[...], preferred_element_type=jnp.float32) + fcb_ref[...]
    m = jnp.max(logits, axis=1, keepdims=True)
    lse = m + jnp.log(jnp.sum(jnp.exp(logits - m), axis=1, keepdims=True))
    out_ref[...] = logits - lse


def _finalize2(q0, q1, h2, aa2, b2, g2, be2, fcW, fcb):
    grid = (N // _ROW_BLK,)
    row = lambda i: (i, 0)
    full = lambda i: (0, 0)
    return pl.pallas_call(
        _finalize2_body,
        grid=grid,
        in_specs=[
            pl.BlockSpec((_ROW_BLK, 80), row),
            pl.BlockSpec((_ROW_BLK, 80), row),
            pl.BlockSpec((_ROW_BLK, HID), row),
            pl.BlockSpec((_ROW_BLK, 16), row),
            pl.BlockSpec((1, HID), full),
            pl.BlockSpec((1, HID), full),
            pl.BlockSpec((1, HID), full),
            pl.BlockSpec((HID, N_CLASSES), full),
            pl.BlockSpec((1, N_CLASSES), full),
        ],
        out_specs=pl.BlockSpec((_ROW_BLK, N_CLASSES), row),
        out_shape=jax.ShapeDtypeStruct((N, N_CLASSES), jnp.float32),
    )(q0, q1, h2, aa2, b2, g2, be2, fcW, fcb)


def _edges_layer1_jnp(h1, aa16, src, dst):
    ex = jnp.exp(_leaky(aa16[src, 0:4] + aa16[dst, 8:12]))
    parts = []
    for c in range(2):
        hcols = h1[:, c * 128:(c + 1) * 128]
        w = jnp.repeat(ex[:, 2 * c:2 * c + 2], HID, axis=1)
        num = jax.ops.segment_sum(hcols[src] * w, dst, num_segments=N)
        den = jax.ops.segment_sum(ex[:, 2 * c:2 * c + 2], dst, num_segments=N)
        parts.append(jnp.concatenate(
            [num, den, jnp.zeros((N, 14), jnp.float32)], axis=1))
    return parts[0], parts[1]


def _edges_layer2_jnp(h2, aa2, src, dst):
    ex = jnp.exp(_leaky(aa2[src, 0:1] + aa2[dst, 8:9]))
    num = jax.ops.segment_sum(h2[src] * ex, dst, num_segments=N)
    den = jax.ops.segment_sum(ex, dst, num_segments=N)
    q0 = jnp.concatenate([num, den, jnp.zeros((N, 15), jnp.float32)], axis=1)
    q1 = jnp.zeros_like(q0)
    return q0, q1


def _att_proj_mat1(att_src1, att_dst1):
    eye = jnp.eye(HEADS, dtype=jnp.float32)
    ms = (att_src1[:, :, None] * eye[:, None, :]).reshape(HEADS * HID, HEADS)
    md = (att_dst1[:, :, None] * eye[:, None, :]).reshape(HEADS * HID, HEADS)
    z = jnp.zeros((HEADS * HID, 4), jnp.float32)
    return jnp.concatenate([ms, z, md, z], axis=1)


def _att_proj_mat2(att_src2, att_dst2):
    z = jnp.zeros((HID, 7), jnp.float32)
    return jnp.concatenate(
        [att_src2.T, z, att_dst2.T, z], axis=1)


def kernel(x, edge_index, W1, att_src1, att_dst1, b1, g1, be1,
           W2, att_src2, att_dst2, b2, g2, be2, fcW, fcb):
    src = edge_index[0]
    dst = edge_index[1]
    S1 = _att_proj_mat1(att_src1, att_dst1)
    S2 = _att_proj_mat2(att_src2, att_dst2)

    h1, aa16 = _proj1(x, W1, S1)
    p0, p1 = _edges_layer1_jnp(h1, aa16, src, dst)
    h2, aa2 = _finalize1(p0, p1, h1, aa16,
                         b1.reshape(1, -1), g1.reshape(1, -1),
                         be1.reshape(1, -1), W2, S2)
    q0, q1 = _edges_layer2_jnp(h2, aa2, src, dst)
    out = _finalize2(q0, q1, h2, aa2,
                     b2.reshape(1, -1), g2.reshape(1, -1),
                     be2.reshape(1, -1), fcW, fcb.reshape(1, -1))
    return out


# speedup vs baseline: 18.6445x; 18.0018x over previous
import functools
import math

import jax
import jax.numpy as jnp
from jax import lax
from jax.experimental import pallas as pl
from jax.experimental.pallas import tpu as pltpu
from jax.experimental.pallas import tpu_sc as plsc

N = 10000
E = 320000
D_IN = 128
HID = 64
HEADS = 4
N_CLASSES = 16

_ROW_BLK = 1000


def _proj1_body(x_ref, w_ref, s_ref, h_ref, aa_ref):
    h = jnp.dot(x_ref[...], w_ref[...], preferred_element_type=jnp.float32)
    h_ref[...] = h
    aa_ref[...] = jnp.dot(h, s_ref[...], preferred_element_type=jnp.float32)


def _proj1(x, W1, S1):
    grid = (N // _ROW_BLK,)
    return pl.pallas_call(
        _proj1_body,
        grid=grid,
        in_specs=[
            pl.BlockSpec((_ROW_BLK, D_IN), lambda i: (i, 0)),
            pl.BlockSpec((D_IN, HEADS * HID), lambda i: (0, 0)),
            pl.BlockSpec((HEADS * HID, 16), lambda i: (0, 0)),
        ],
        out_specs=[
            pl.BlockSpec((_ROW_BLK, HEADS * HID), lambda i: (i, 0)),
            pl.BlockSpec((_ROW_BLK, 16), lambda i: (i, 0)),
        ],
        out_shape=[
            jax.ShapeDtypeStruct((N, HEADS * HID), jnp.float32),
            jax.ShapeDtypeStruct((N, 16), jnp.float32),
        ],
    )(x, W1, S1)


def _leaky(x):
    return jnp.where(x > 0, x, 0.2 * x)


def _elu(x):
    return jnp.where(x > 0, x, jnp.exp(x) - 1.0)


def _finalize1_body(p0_ref, p1_ref, h1_ref, aa_ref, b1_ref, g1_ref, be1_ref,
                    w2_ref, s2_ref, h2_ref, aa2_ref):
    aa = aa_ref[...]
    ex_ii = jnp.exp(_leaky(aa[:, 0:4] + aa[:, 8:12]))
    h1 = h1_ref[...]
    cols = []
    for h in range(HEADS):
        part = p0_ref[...] if h < 2 else p1_ref[...]
        j = h % 2
        num = part[:, j * HID:(j + 1) * HID] + ex_ii[:, h:h + 1] * h1[:, h * HID:(h + 1) * HID]
        den = part[:, 128 + j:129 + j] + ex_ii[:, h:h + 1]
        cols.append(num / (den + 1e-16))
    agg = jnp.concatenate(cols, axis=1) + b1_ref[...]
    scale = g1_ref[...] * (1.0 / math.sqrt(1.0 + 1e-5))
    act = _elu(agg * scale + be1_ref[...])
    h2 = jnp.dot(act, w2_ref[...], preferred_element_type=jnp.float32)
    h2_ref[...] = h2
    aa2_ref[...] = jnp.dot(h2, s2_ref[...], preferred_element_type=jnp.float32)


def _finalize1(p0, p1, h1, aa16, b1, g1, be1, W2, S2):
    F = HEADS * HID
    grid = (N // _ROW_BLK,)
    row = lambda i: (i, 0)
    full = lambda i: (0, 0)
    return pl.pallas_call(
        _finalize1_body,
        grid=grid,
        in_specs=[
            pl.BlockSpec((_ROW_BLK, 144), row),
            pl.BlockSpec((_ROW_BLK, 144), row),
            pl.BlockSpec((_ROW_BLK, F), row),
            pl.BlockSpec((_ROW_BLK, 16), row),
            pl.BlockSpec((1, F), full),
            pl.BlockSpec((1, F), full),
            pl.BlockSpec((1, F), full),
            pl.BlockSpec((F, HID), full),
            pl.BlockSpec((HID, 16), full),
        ],
        out_specs=[
            pl.BlockSpec((_ROW_BLK, HID), row),
            pl.BlockSpec((_ROW_BLK, 16), row),
        ],
        out_shape=[
            jax.ShapeDtypeStruct((N, HID), jnp.float32),
            jax.ShapeDtypeStruct((N, 16), jnp.float32),
        ],
    )(p0, p1, h1, aa16, b1, g1, be1, W2, S2)


def _finalize2_body(q0_ref, q1_ref, h2_ref, aa2_ref, b2_ref, g2_ref, be2_ref,
                    fcw_ref, fcb_ref, out_ref):
    aa2 = aa2_ref[...]
    ex_ii = jnp.exp(_leaky(aa2[:, 0:1] + aa2[:, 8:9]))
    h2 = h2_ref[...]
    q0 = q0_ref[...]
    q1 = q1_ref[...]
    num = q0[:, :HID] + q1[:, :HID] + ex_ii * h2
    den = q0[:, HID:HID + 1] + q1[:, HID:HID + 1] + ex_ii
    agg = num / (den + 1e-16) + b2_ref[...]
    scale = g2_ref[...] * (1.0 / math.sqrt(1.0 + 1e-5))
    act = _elu(agg * scale + be2_ref[...])
    logits = jnp.dot(act, fcw_ref[...], preferred_element_type=jnp.float32) + fcb_ref[...]
    m = jnp.max(logits, axis=1, keepdims=True)
    lse = m + jnp.log(jnp.sum(jnp.exp(logits - m), axis=1, keepdims=True))
    out_ref[...] = logits - lse


def _finalize2(q0, q1, h2, aa2, b2, g2, be2, fcW, fcb):
    grid = (N // _ROW_BLK,)
    row = lambda i: (i, 0)
    full = lambda i: (0, 0)
    return pl.pallas_call(
        _finalize2_body,
        grid=grid,
        in_specs=[
            pl.BlockSpec((_ROW_BLK, 80), row),
            pl.BlockSpec((_ROW_BLK, 80), row),
            pl.BlockSpec((_ROW_BLK, HID), row),
            pl.BlockSpec((_ROW_BLK, 16), row),
            pl.BlockSpec((1, HID), full),
            pl.BlockSpec((1, HID), full),
            pl.BlockSpec((1, HID), full),
            pl.BlockSpec((HID, N_CLASSES), full),
            pl.BlockSpec((1, N_CLASSES), full),
        ],
        out_specs=pl.BlockSpec((_ROW_BLK, N_CLASSES), row),
        out_shape=jax.ShapeDtypeStruct((N, N_CLASSES), jnp.float32),
    )(q0, q1, h2, aa2, b2, g2, be2, fcW, fcb)


_N16 = 10112
_CH = 128


def _make_sc_edges(F, acc_w, nhl, eps, dup, name):
    mesh = plsc.VectorSubcoreMesh(core_axis_name="c", subcore_axis_name="s")
    chunks = eps // _CH
    fo = F // nhl
    nv = fo // 16

    scratch = [
        pltpu.VMEM_SHARED((_N16, acc_w), jnp.float32),
        pltpu.VMEM((_CH,), jnp.int32),
        pltpu.VMEM((_CH,), jnp.int32),
        pltpu.VMEM((_CH, F), jnp.float32),
        pltpu.VMEM((_CH, acc_w), jnp.float32),
        pltpu.SemaphoreType.DMA,
    ]
    for _ in range(nhl):
        scratch += [pltpu.VMEM((_CH,), jnp.float32),
                    pltpu.VMEM((_CH,), jnp.float32),
                    pltpu.VMEM((_CH + 16,), jnp.float32)]
    if dup:
        for _ in range(nhl):
            scratch += [pltpu.VMEM((_CH,), jnp.int32),
                        pltpu.VMEM((_CH,), jnp.int32)]

    @functools.partial(
        pl.kernel,
        out_type=jax.ShapeDtypeStruct((2 * N, acc_w), jnp.float32),
        mesh=mesh,
        scratch_types=scratch,
        compiler_params=pltpu.CompilerParams(use_tc_tiling_on_sc=False),
        name=name,
    )
    def k(tab, aas_hbm, aad_hbm, src_hbm, dst_hbm, out_hbm,
          acc, srcbuf, dstbuf, hbuf, prod, sem, *bufs):
        asb = [bufs[3 * t] for t in range(nhl)]
        adb = [bufs[3 * t + 1] for t in range(nhl)]
        exb = [bufs[3 * t + 2] for t in range(nhl)]
        if dup:
            idxS = [bufs[3 * nhl + 2 * t] for t in range(nhl)]
            idxD = [bufs[3 * nhl + 2 * t + 1] for t in range(nhl)]
        c = lax.axis_index("c")
        s = lax.axis_index("s")
        lanes = jnp.arange(16, dtype=jnp.int32)
        zero16 = jnp.zeros((16,), jnp.float32)

        def zrow(r, carry):
            for q in range(acc_w // 16):
                prod[r, pl.ds(16 * q, 16)] = zero16
            return carry
        lax.fori_loop(0, _CH, zrow, 0)
        zb = s * (_N16 // 16)
        for t in range(4):
            pltpu.sync_copy(prod, acc.at[pl.ds(zb + 128 * t, 128)])
        zrem = _N16 // 16 - 512
        pltpu.sync_copy(prod.at[pl.ds(0, zrem)],
                        acc.at[pl.ds(zb + 512, zrem)])
        plsc.subcore_barrier()

        if dup:
            ebase = s * eps
        else:
            ebase = (c * 16 + s) * eps

        def chunk(j, carry):
            base = ebase + j * _CH
            pltpu.sync_copy(src_hbm.at[pl.ds(base, _CH)], srcbuf)
            pltpu.sync_copy(dst_hbm.at[pl.ds(base, _CH)], dstbuf)
            if dup:
                offs = c * N
                offd = c * _N16
                for q in range(_CH // 16):
                    sl = pl.ds(16 * q, 16)
                    sv = srcbuf[sl] + offs
                    srcbuf[sl] = sv
                    dv = dstbuf[sl] + offd
                    for hl in range(nhl):
                        idxS[hl][sl] = sv * 2 + hl
                        idxD[hl][sl] = dv * 2 + hl
                cps = [pltpu.async_copy(aas_hbm.at[idxS[hl]], asb[hl], sem)
                       for hl in range(nhl)]
                cps += [pltpu.async_copy(aad_hbm.at[idxD[hl]], adb[hl], sem)
                        for hl in range(nhl)]
            else:
                cps = [pltpu.async_copy(aas_hbm.at[srcbuf], asb[0], sem),
                       pltpu.async_copy(aad_hbm.at[dstbuf], adb[0], sem)]
            cps.append(pltpu.async_copy(tab.at[srcbuf], hbuf, sem))
            for cp in cps:
                cp.wait()
            for hl in range(nhl):
                for q in range(_CH // 16):
                    sl = pl.ds(16 * q, 16)
                    e = asb[hl][sl] + adb[hl][sl]
                    e = jnp.where(e > 0, e, 0.2 * e)
                    exb[hl][sl] = jnp.exp(e)
            def edge(i, carry2):
                evs = []
                for hl in range(nhl):
                    ev = jnp.full((16,), exb[hl][pl.ds(i, 16)][0], jnp.float32)
                    evs.append(ev)
                    for q in range(nv):
                        cix = hl * fo + 16 * q
                        prod[i, pl.ds(cix, 16)] = hbuf[i, pl.ds(cix, 16)] * ev
                if nhl == 2:
                    tail = jnp.where(lanes == 0, evs[0],
                                     jnp.where(lanes == 1, evs[1], zero16))
                else:
                    tail = jnp.where(lanes == 0, evs[0], zero16)
                prod[i, pl.ds(F, 16)] = tail
                return carry2
            lax.fori_loop(0, _CH, edge, 0)
            pltpu.sync_copy(prod, acc.at[dstbuf], add=True)
            return carry
        lax.fori_loop(0, chunks, chunk, 0)
        plsc.subcore_barrier()

        for t in range(5):
            b = t * 16 + s
            r0 = b * 128

            @pl.when(b <= (N // 128) - 1)
            def _():
                pltpu.sync_copy(acc.at[pl.ds(r0, 128)], prod)
                pltpu.sync_copy(prod, out_hbm.at[pl.ds(c * N + r0, 128)])

            @pl.when(b == N // 128)
            def _():
                rem = N % 128
                pltpu.sync_copy(acc.at[pl.ds(r0, rem)], prod.at[pl.ds(0, rem)])
                pltpu.sync_copy(prod.at[pl.ds(0, rem)],
                                out_hbm.at[pl.ds(c * N + r0, rem)])

    return k


_EPS1 = ((E + 16 * _CH - 1) // (16 * _CH)) * _CH
_EPS2 = ((E + 32 * _CH - 1) // (32 * _CH)) * _CH

_sc_edges1 = _make_sc_edges(F=128, acc_w=144, nhl=2, eps=_EPS1, dup=1,
                            name="gat_edges_l1")
_sc_edges2 = _make_sc_edges(F=64, acc_w=80, nhl=1, eps=_EPS2, dup=0,
                            name="gat_edges_l2")


def _edges_layer1_jnp(h1, aa16, src, dst):
    ex = jnp.exp(_leaky(aa16[src, 0:4] + aa16[dst, 8:12]))
    parts = []
    for c in range(2):
        hcols = h1[:, c * 128:(c + 1) * 128]
        w = jnp.repeat(ex[:, 2 * c:2 * c + 2], HID, axis=1)
        num = jax.ops.segment_sum(hcols[src] * w, dst, num_segments=N)
        den = jax.ops.segment_sum(ex[:, 2 * c:2 * c + 2], dst, num_segments=N)
        parts.append(jnp.concatenate(
            [num, den, jnp.zeros((N, 14), jnp.float32)], axis=1))
    return parts[0], parts[1]


def _edges_layer2_jnp(h2, aa2, src, dst):
    ex = jnp.exp(_leaky(aa2[src, 0:1] + aa2[dst, 8:9]))
    num = jax.ops.segment_sum(h2[src] * ex, dst, num_segments=N)
    den = jax.ops.segment_sum(ex, dst, num_segments=N)
    q0 = jnp.concatenate([num, den, jnp.zeros((N, 15), jnp.float32)], axis=1)
    q1 = jnp.zeros_like(q0)
    return q0, q1


def _att_proj_mat1(att_src1, att_dst1):
    eye = jnp.eye(HEADS, dtype=jnp.float32)
    ms = (att_src1[:, :, None] * eye[:, None, :]).reshape(HEADS * HID, HEADS)
    md = (att_dst1[:, :, None] * eye[:, None, :]).reshape(HEADS * HID, HEADS)
    z = jnp.zeros((HEADS * HID, 4), jnp.float32)
    return jnp.concatenate([ms, z, md, z], axis=1)


def _att_proj_mat2(att_src2, att_dst2):
    z = jnp.zeros((HID, 7), jnp.float32)
    return jnp.concatenate(
        [att_src2.T, z, att_dst2.T, z], axis=1)


def kernel(x, edge_index, W1, att_src1, att_dst1, b1, g1, be1,
           W2, att_src2, att_dst2, b2, g2, be2, fcW, fcb):
    src = edge_index[0]
    dst = edge_index[1]
    S1 = _att_proj_mat1(att_src1, att_dst1)
    S2 = _att_proj_mat2(att_src2, att_dst2)

    pad1 = 16 * _EPS1 - E
    srcp1 = jnp.concatenate([src, jnp.zeros((pad1,), jnp.int32)])
    dstp1 = jnp.concatenate([dst, jnp.full((pad1,), N, jnp.int32)])
    pad2 = 32 * _EPS2 - E
    srcp2 = jnp.concatenate([src, jnp.zeros((pad2,), jnp.int32)])
    dstp2 = jnp.concatenate([dst, jnp.full((pad2,), N, jnp.int32)])

    h1, aa16 = _proj1(x, W1, S1)
    hcat1 = h1.reshape(N, 2, 128).transpose(1, 0, 2).reshape(2 * N, 128)
    aaS1 = aa16[:, 0:4].reshape(N, 2, 2).transpose(1, 0, 2).reshape(-1)
    adst_p = jnp.concatenate(
        [aa16[:, 8:12], jnp.zeros((_N16 - N, 4), jnp.float32)])
    aaD1 = adst_p.reshape(_N16, 2, 2).transpose(1, 0, 2).reshape(-1)
    out1 = _sc_edges1(hcat1, aaS1, aaD1, srcp1, dstp1)
    p0, p1 = out1[:N], out1[N:]
    h2, aa2 = _finalize1(p0, p1, h1, aa16,
                         b1.reshape(1, -1), g1.reshape(1, -1),
                         be1.reshape(1, -1), W2, S2)
    aaS2 = aa2[:, 0]
    aaD2 = jnp.concatenate([aa2[:, 8], jnp.zeros((_N16 - N,), jnp.float32)])
    out2 = _sc_edges2(h2, aaS2, aaD2, srcp2, dstp2)
    q0, q1 = out2[:N], out2[N:]
    out = _finalize2(q0, q1, h2, aa2,
                     b2.reshape(1, -1), g2.reshape(1, -1),
                     be2.reshape(1, -1), fcW, fcb.reshape(1, -1))
    return out


# speedup vs baseline: 24.0054x; 1.2875x over previous
import functools
import math

import jax
import jax.numpy as jnp
from jax import lax
from jax.experimental import pallas as pl
from jax.experimental.pallas import tpu as pltpu
from jax.experimental.pallas import tpu_sc as plsc

N = 10000
E = 320000
D_IN = 128
HID = 64
HEADS = 4
N_CLASSES = 16

_ROW_BLK = 1000


def _proj1_body(x_ref, w_ref, s_ref, h_ref, aa_ref):
    h = jnp.dot(x_ref[...], w_ref[...], preferred_element_type=jnp.float32)
    h_ref[...] = h
    aa_ref[...] = jnp.dot(h, s_ref[...], preferred_element_type=jnp.float32)


def _proj1(x, W1, S1):
    grid = (N // _ROW_BLK,)
    return pl.pallas_call(
        _proj1_body,
        grid=grid,
        in_specs=[
            pl.BlockSpec((_ROW_BLK, D_IN), lambda i: (i, 0)),
            pl.BlockSpec((D_IN, HEADS * HID), lambda i: (0, 0)),
            pl.BlockSpec((HEADS * HID, 16), lambda i: (0, 0)),
        ],
        out_specs=[
            pl.BlockSpec((_ROW_BLK, HEADS * HID), lambda i: (i, 0)),
            pl.BlockSpec((_ROW_BLK, 16), lambda i: (i, 0)),
        ],
        out_shape=[
            jax.ShapeDtypeStruct((N, HEADS * HID), jnp.float32),
            jax.ShapeDtypeStruct((N, 16), jnp.float32),
        ],
    )(x, W1, S1)


def _leaky(x):
    return jnp.where(x > 0, x, 0.2 * x)


def _elu(x):
    return jnp.where(x > 0, x, jnp.exp(x) - 1.0)


def _finalize1_body(p0_ref, p1_ref, h1_ref, aa_ref, b1_ref, g1_ref, be1_ref,
                    w2_ref, s2_ref, h2_ref, aa2_ref):
    aa = aa_ref[...]
    ex_ii = jnp.exp(_leaky(aa[:, 0:4] + aa[:, 8:12]))
    h1 = h1_ref[...]
    cols = []
    for h in range(HEADS):
        part = p0_ref[...] if h < 2 else p1_ref[...]
        j = h % 2
        num = part[:, j * HID:(j + 1) * HID] + ex_ii[:, h:h + 1] * h1[:, h * HID:(h + 1) * HID]
        den = part[:, 128 + j:129 + j] + ex_ii[:, h:h + 1]
        cols.append(num / (den + 1e-16))
    agg = jnp.concatenate(cols, axis=1) + b1_ref[...]
    scale = g1_ref[...] * (1.0 / math.sqrt(1.0 + 1e-5))
    act = _elu(agg * scale + be1_ref[...])
    h2 = jnp.dot(act, w2_ref[...], preferred_element_type=jnp.float32)
    h2_ref[...] = h2
    aa2_ref[...] = jnp.dot(h2, s2_ref[...], preferred_element_type=jnp.float32)


def _finalize1(p0, p1, h1, aa16, b1, g1, be1, W2, S2):
    F = HEADS * HID
    grid = (N // _ROW_BLK,)
    row = lambda i: (i, 0)
    full = lambda i: (0, 0)
    return pl.pallas_call(
        _finalize1_body,
        grid=grid,
        in_specs=[
            pl.BlockSpec((_ROW_BLK, 144), row),
            pl.BlockSpec((_ROW_BLK, 144), row),
            pl.BlockSpec((_ROW_BLK, F), row),
            pl.BlockSpec((_ROW_BLK, 16), row),
            pl.BlockSpec((1, F), full),
            pl.BlockSpec((1, F), full),
            pl.BlockSpec((1, F), full),
            pl.BlockSpec((F, HID), full),
            pl.BlockSpec((HID, 16), full),
        ],
        out_specs=[
            pl.BlockSpec((_ROW_BLK, HID), row),
            pl.BlockSpec((_ROW_BLK, 16), row),
        ],
        out_shape=[
            jax.ShapeDtypeStruct((N, HID), jnp.float32),
            jax.ShapeDtypeStruct((N, 16), jnp.float32),
        ],
    )(p0, p1, h1, aa16, b1, g1, be1, W2, S2)


def _finalize2_body(q0_ref, q1_ref, h2_ref, aa2_ref, b2_ref, g2_ref, be2_ref,
                    fcw_ref, fcb_ref, out_ref):
    aa2 = aa2_ref[...]
    ex_ii = jnp.exp(_leaky(aa2[:, 0:1] + aa2[:, 8:9]))
    h2 = h2_ref[...]
    q0 = q0_ref[...]
    q1 = q1_ref[...]
    num = q0[:, :HID] + q1[:, :HID] + ex_ii * h2
    den = q0[:, HID:HID + 1] + q1[:, HID:HID + 1] + ex_ii
    agg = num / (den + 1e-16) + b2_ref[...]
    scale = g2_ref[...] * (1.0 / math.sqrt(1.0 + 1e-5))
    act = _elu(agg * scale + be2_ref[...])
    logits = jnp.dot(act, fcw_ref[...], preferred_element_type=jnp.float32) + fcb_ref[...]
    m = jnp.max(logits, axis=1, keepdims=True)
    lse = m + jnp.log(jnp.sum(jnp.exp(logits - m), axis=1, keepdims=True))
    out_ref[...] = logits - lse


def _finalize2(q0, q1, h2, aa2, b2, g2, be2, fcW, fcb):
    grid = (N // _ROW_BLK,)
    row = lambda i: (i, 0)
    full = lambda i: (0, 0)
    return pl.pallas_call(
        _finalize2_body,
        grid=grid,
        in_specs=[
            pl.BlockSpec((_ROW_BLK, 80), row),
            pl.BlockSpec((_ROW_BLK, 80), row),
            pl.BlockSpec((_ROW_BLK, HID), row),
            pl.BlockSpec((_ROW_BLK, 16), row),
            pl.BlockSpec((1, HID), full),
            pl.BlockSpec((1, HID), full),
            pl.BlockSpec((1, HID), full),
            pl.BlockSpec((HID, N_CLASSES), full),
            pl.BlockSpec((1, N_CLASSES), full),
        ],
        out_specs=pl.BlockSpec((_ROW_BLK, N_CLASSES), row),
        out_shape=jax.ShapeDtypeStruct((N, N_CLASSES), jnp.float32),
    )(q0, q1, h2, aa2, b2, g2, be2, fcW, fcb)


_N16 = 10112
_CH = 128


def _make_sc_edges(F, acc_w, nhl, eps, dup, ch, name):
    mesh = plsc.VectorSubcoreMesh(core_axis_name="c", subcore_axis_name="s")
    chunks = eps // ch
    assert chunks % 2 == 0
    fo = F // nhl
    nv = fo // 16

    scratch = [pltpu.VMEM_SHARED((_N16, acc_w), jnp.float32)]
    for _ in range(2):
        scratch += [pltpu.VMEM((ch,), jnp.int32),
                    pltpu.VMEM((ch,), jnp.int32),
                    pltpu.VMEM((ch,), jnp.int32),
                    pltpu.VMEM((ch, F), jnp.float32),
                    pltpu.VMEM((ch, acc_w), jnp.float32)]
        for _ in range(nhl):
            scratch += [pltpu.VMEM((ch,), jnp.float32),
                        pltpu.VMEM((ch,), jnp.float32),
                        pltpu.VMEM((ch + 16,), jnp.float32)]
        if dup:
            for _ in range(nhl):
                scratch += [pltpu.VMEM((ch,), jnp.int32),
                            pltpu.VMEM((ch,), jnp.int32)]
    scratch += [pltpu.SemaphoreType.DMA] * 4

    @functools.partial(
        pl.kernel,
        out_type=jax.ShapeDtypeStruct((2 * N, acc_w), jnp.float32),
        mesh=mesh,
        scratch_types=scratch,
        compiler_params=pltpu.CompilerParams(use_tc_tiling_on_sc=False),
        name=name,
    )
    def k(tab, aas_hbm, aad_hbm, src_hbm, dst_hbm, out_hbm, acc, *bufs):
        it = iter(bufs)
        srcb, dstb, scix, hbuf, prod = ([None, None] for _ in range(5))
        asb = [[None] * nhl, [None] * nhl]
        adb = [[None] * nhl, [None] * nhl]
        exb = [[None] * nhl, [None] * nhl]
        idxS = [[None] * nhl, [None] * nhl]
        idxD = [[None] * nhl, [None] * nhl]
        for p in range(2):
            srcb[p] = next(it); dstb[p] = next(it); scix[p] = next(it)
            hbuf[p] = next(it); prod[p] = next(it)
            for hl in range(nhl):
                asb[p][hl] = next(it); adb[p][hl] = next(it); exb[p][hl] = next(it)
            if dup:
                for hl in range(nhl):
                    idxS[p][hl] = next(it); idxD[p][hl] = next(it)
        sem_g = [next(it), next(it)]
        sem_s = [next(it), next(it)]

        c = lax.axis_index("c")
        s = lax.axis_index("s")
        lanes = jnp.arange(16, dtype=jnp.int32)
        zero16 = jnp.zeros((16,), jnp.float32)

        def zrow(r, carry):
            for q in range(acc_w // 16):
                prod[0][r, pl.ds(16 * q, 16)] = zero16
            return carry
        lax.fori_loop(0, ch, zrow, 0)
        zb = s * (_N16 // 16)
        znf = (_N16 // 16) // ch
        zrem = (_N16 // 16) % ch
        for t in range(znf):
            pltpu.sync_copy(prod[0], acc.at[pl.ds(zb + ch * t, ch)])
        if zrem:
            pltpu.sync_copy(prod[0].at[pl.ds(0, zrem)],
                            acc.at[pl.ds(zb + ch * znf, zrem)])
        plsc.subcore_barrier()

        if dup:
            ebase = s * eps
        else:
            ebase = (c * 16 + s) * eps

        def fetch_and_issue(p, jn):
            base = ebase + jn * ch
            pltpu.sync_copy(src_hbm.at[pl.ds(base, ch)], srcb[p])
            pltpu.sync_copy(dst_hbm.at[pl.ds(base, ch)], dstb[p])
            if dup:
                offs = c * N
                offd = c * _N16
                for q in range(ch // 16):
                    sl = pl.ds(16 * q, 16)
                    sv = srcb[p][sl] + offs
                    srcb[p][sl] = sv
                    dv = dstb[p][sl] + offd
                    for hl in range(nhl):
                        idxS[p][hl][sl] = sv * 2 + hl
                        idxD[p][hl][sl] = dv * 2 + hl
                for hl in range(nhl):
                    pltpu.async_copy(aas_hbm.at[idxS[p][hl]], asb[p][hl], sem_g[p])
                    pltpu.async_copy(aad_hbm.at[idxD[p][hl]], adb[p][hl], sem_g[p])
            else:
                pltpu.async_copy(aas_hbm.at[srcb[p]], asb[p][0], sem_g[p])
                pltpu.async_copy(aad_hbm.at[dstb[p]], adb[p][0], sem_g[p])
            pltpu.async_copy(tab.at[srcb[p]], hbuf[p], sem_g[p])

        def wait_gathers(p):
            for hl in range(nhl):
                pltpu.make_async_copy(
                    aas_hbm.at[pl.ds(0, ch)], asb[p][hl], sem_g[p]).wait()
                pltpu.make_async_copy(
                    aad_hbm.at[pl.ds(0, ch)], adb[p][hl], sem_g[p]).wait()
            pltpu.make_async_copy(
                tab.at[pl.ds(0, ch)], hbuf[p], sem_g[p]).wait()

        def compute(p):
            for hl in range(nhl):
                for q in range(ch // 16):
                    sl = pl.ds(16 * q, 16)
                    e = asb[p][hl][sl] + adb[p][hl][sl]
                    e = jnp.where(e > 0, e, 0.2 * e)
                    exb[p][hl][sl] = jnp.exp(e)
            for q in range(ch // 16):
                sl = pl.ds(16 * q, 16)
                scix[p][sl] = dstb[p][sl]

            def edge2(t, carry2):
                for u in range(2):
                    i = t * 2 + u
                    evs = []
                    for hl in range(nhl):
                        ev = jnp.full(
                            (16,), exb[p][hl][pl.ds(i, 16)][0], jnp.float32)
                        evs.append(ev)
                        for q in range(nv):
                            cix = hl * fo + 16 * q
                            prod[p][i, pl.ds(cix, 16)] = (
                                hbuf[p][i, pl.ds(cix, 16)] * ev)
                    if nhl == 2:
                        tail = jnp.where(lanes == 0, evs[0],
                                         jnp.where(lanes == 1, evs[1], zero16))
                    else:
                        tail = jnp.where(lanes == 0, evs[0], zero16)
                    prod[p][i, pl.ds(F, 16)] = tail
                return carry2
            lax.fori_loop(0, ch // 2, edge2, 0)

        def do_scatter(p):
            pltpu.async_copy(prod[p], acc.at[scix[p]], sem_s[p], add=True)

        def wait_scatter(p):
            pltpu.make_async_copy(prod[p], acc.at[pl.ds(0, ch)],
                                  sem_s[p]).wait()

        fetch_and_issue(0, 0)

        def body(i2, carry):
            for p in range(2):
                j = 2 * i2 + p
                jn = j + 1

                @pl.when(jn < chunks)
                def _():
                    fetch_and_issue(1 - p, jn)
                wait_gathers(p)

                @pl.when(j >= 2)
                def _():
                    wait_scatter(p)
                compute(p)
                do_scatter(p)
            return carry
        lax.fori_loop(0, chunks // 2, body, 0)
        wait_scatter(0)
        wait_scatter(1)
        plsc.subcore_barrier()

        onf = N // ch
        orem = N % ch
        for t in range((onf + 1 + 15) // 16):
            b = t * 16 + s
            r0 = b * ch

            @pl.when(b <= onf - 1)
            def _():
                pltpu.sync_copy(acc.at[pl.ds(r0, ch)], prod[0])
                pltpu.sync_copy(prod[0], out_hbm.at[pl.ds(c * N + r0, ch)])

            if orem:
                @pl.when(b == onf)
                def _():
                    pltpu.sync_copy(acc.at[pl.ds(r0, orem)],
                                    prod[0].at[pl.ds(0, orem)])
                    pltpu.sync_copy(prod[0].at[pl.ds(0, orem)],
                                    out_hbm.at[pl.ds(c * N + r0, orem)])

    return k


def _round_up(x, m):
    return ((x + m - 1) // m) * m


_EPS1 = _round_up((E + 16 * _CH - 1) // (16 * _CH), 2) * _CH
_EPS2 = _round_up((E + 32 * _CH - 1) // (32 * _CH), 2) * _CH

_sc_edges1 = _make_sc_edges(F=128, acc_w=144, nhl=2, eps=_EPS1, dup=1,
                            ch=64, name="gat_edges_l1")
_sc_edges2 = _make_sc_edges(F=64, acc_w=80, nhl=1, eps=_EPS2, dup=0,
                            ch=128, name="gat_edges_l2")


def _edges_layer1_jnp(h1, aa16, src, dst):
    ex = jnp.exp(_leaky(aa16[src, 0:4] + aa16[dst, 8:12]))
    parts = []
    for c in range(2):
        hcols = h1[:, c * 128:(c + 1) * 128]
        w = jnp.repeat(ex[:, 2 * c:2 * c + 2], HID, axis=1)
        num = jax.ops.segment_sum(hcols[src] * w, dst, num_segments=N)
        den = jax.ops.segment_sum(ex[:, 2 * c:2 * c + 2], dst, num_segments=N)
        parts.append(jnp.concatenate(
            [num, den, jnp.zeros((N, 14), jnp.float32)], axis=1))
    return parts[0], parts[1]


def _edges_layer2_jnp(h2, aa2, src, dst):
    ex = jnp.exp(_leaky(aa2[src, 0:1] + aa2[dst, 8:9]))
    num = jax.ops.segment_sum(h2[src] * ex, dst, num_segments=N)
    den = jax.ops.segment_sum(ex, dst, num_segments=N)
    q0 = jnp.concatenate([num, den, jnp.zeros((N, 15), jnp.float32)], axis=1)
    q1 = jnp.zeros_like(q0)
    return q0, q1


def _att_proj_mat1(att_src1, att_dst1):
    eye = jnp.eye(HEADS, dtype=jnp.float32)
    ms = (att_src1[:, :, None] * eye[:, None, :]).reshape(HEADS * HID, HEADS)
    md = (att_dst1[:, :, None] * eye[:, None, :]).reshape(HEADS * HID, HEADS)
    z = jnp.zeros((HEADS * HID, 4), jnp.float32)
    return jnp.concatenate([ms, z, md, z], axis=1)


def _att_proj_mat2(att_src2, att_dst2):
    z = jnp.zeros((HID, 7), jnp.float32)
    return jnp.concatenate(
        [att_src2.T, z, att_dst2.T, z], axis=1)


def kernel(x, edge_index, W1, att_src1, att_dst1, b1, g1, be1,
           W2, att_src2, att_dst2, b2, g2, be2, fcW, fcb):
    src = edge_index[0]
    dst = edge_index[1]
    S1 = _att_proj_mat1(att_src1, att_dst1)
    S2 = _att_proj_mat2(att_src2, att_dst2)

    pad1 = 16 * _EPS1 - E
    srcp1 = jnp.concatenate([src, jnp.zeros((pad1,), jnp.int32)])
    dstp1 = jnp.concatenate([dst, jnp.full((pad1,), N, jnp.int32)])
    pad2 = 32 * _EPS2 - E
    srcp2 = jnp.concatenate([src, jnp.zeros((pad2,), jnp.int32)])
    dstp2 = jnp.concatenate([dst, jnp.full((pad2,), N, jnp.int32)])

    h1, aa16 = _proj1(x, W1, S1)
    hcat1 = h1.reshape(N, 2, 128).transpose(1, 0, 2).reshape(2 * N, 128)
    aaS1 = aa16[:, 0:4].reshape(N, 2, 2).transpose(1, 0, 2).reshape(-1)
    adst_p = jnp.concatenate(
        [aa16[:, 8:12], jnp.zeros((_N16 - N, 4), jnp.float32)])
    aaD1 = adst_p.reshape(_N16, 2, 2).transpose(1, 0, 2).reshape(-1)
    out1 = _sc_edges1(hcat1, aaS1, aaD1, srcp1, dstp1)
    p0, p1 = out1[:N], out1[N:]
    h2, aa2 = _finalize1(p0, p1, h1, aa16,
                         b1.reshape(1, -1), g1.reshape(1, -1),
                         be1.reshape(1, -1), W2, S2)
    aaS2 = aa2[:, 0]
    aaD2 = jnp.concatenate([aa2[:, 8], jnp.zeros((_N16 - N,), jnp.float32)])
    out2 = _sc_edges2(h2, aaS2, aaD2, srcp2, dstp2)
    q0, q1 = out2[:N], out2[N:]
    out = _finalize2(q0, q1, h2, aa2,
                     b2.reshape(1, -1), g2.reshape(1, -1),
                     be2.reshape(1, -1), fcW, fcb.reshape(1, -1))
    return out
